# baseline (speedup 1.0000x reference)
import functools

import jax
import jax.numpy as jnp
from jax import lax
from jax.experimental import pallas as pl
from jax.experimental.pallas import tpu as pltpu
from jax.experimental.pallas import tpu_sc as plsc

NC = 2
NS = 16
L = 16
NW = NC * NS

BATCH = 16384
D = 32
B_PER_W = BATCH // NW
CHUNK = 128
NCHUNK = B_PER_W // CHUNK
GPC = CHUNK // L


def _body(samp_hbm, user_hbm, book_hbm, out_hbm,
          samp_v, idx_u, idx_b, u_rows, b_rows, out_v, sem):
  wid = lax.axis_index("s") * NC + lax.axis_index("c")
  base = wid * B_PER_W

  pltpu.sync_copy(samp_hbm.at[pl.ds(base * 2, 2 * B_PER_W)], samp_v)

  iota = lax.iota(jnp.int32, L)
  for j in range(NCHUNK):
    for gg in range(GPC):
      g = j * GPC + gg
      pos = 2 * L * g + 2 * iota
      idx_u[j, pl.ds(gg * L, L)] = plsc.load_gather(samp_v, [pos])
      idx_b[j, pl.ds(gg * L, L)] = plsc.load_gather(samp_v, [pos + 1])

  copies = []
  for j in range(NCHUNK):
    copies.append(pltpu.async_copy(user_hbm.at[idx_u.at[j]], u_rows.at[j], sem))
    copies.append(pltpu.async_copy(book_hbm.at[idx_b.at[j]], b_rows.at[j], sem))
  for c in copies:
    c.wait()

  for j in range(NCHUNK):
    uc = u_rows.at[j]
    bc = b_rows.at[j]
    for gg in range(GPC):
      row = gg * L + iota
      acc = jnp.zeros((L,), jnp.float32)
      for d in range(D):
        dv = jnp.full((L,), d, jnp.int32)
        acc = acc + plsc.load_gather(uc, [row, dv]) * plsc.load_gather(bc, [row, dv])
      res = 5.5 / (1.0 + jnp.exp(-acc))
      out_v[pl.ds((j * GPC + gg) * L, L)] = res

  pltpu.sync_copy(out_v, out_hbm.at[pl.ds(base, B_PER_W)])


def _make_kernel(interpret=False):
  mesh = plsc.VectorSubcoreMesh(
      core_axis_name="c", subcore_axis_name="s",
      num_cores=NC, num_subcores=NS)
  return pl.kernel(
      _body,
      out_type=jax.ShapeDtypeStruct((BATCH,), jnp.float32),
      mesh=mesh,
      scratch_types=[
          pltpu.VMEM((2 * B_PER_W,), jnp.int32),
          pltpu.VMEM((NCHUNK, CHUNK), jnp.int32),
          pltpu.VMEM((NCHUNK, CHUNK), jnp.int32),
          pltpu.VMEM((NCHUNK, CHUNK, D), jnp.float32),
          pltpu.VMEM((NCHUNK, CHUNK, D), jnp.float32),
          pltpu.VMEM((B_PER_W,), jnp.float32),
          pltpu.SemaphoreType.DMA,
      ],
      compiler_params=pltpu.CompilerParams(
          needs_layout_passes=False, use_tc_tiling_on_sc=False),
      interpret=interpret,
  )


@jax.jit
def kernel(samples, user_embedding, book_embedding):
  samp_flat = samples.reshape(-1).astype(jnp.int32)
  return _make_kernel()(samp_flat, user_embedding, book_embedding)
